# Initial kernel scaffold; baseline (speedup 1.0000x reference)
import jax
import jax.numpy as jnp
from jax.experimental import pallas as pl


def kernel(x, W1, W2, W3, W4, W5, W6, W8, W9, W10, W11):
    raise NotImplementedError("write your pallas kernel here")


# speedup vs baseline: 6.0395x; 6.0395x over previous
import functools

import jax
import jax.numpy as jnp
from jax.experimental import pallas as pl
from jax.experimental.pallas import tpu as pltpu

N = 2048
K = 20
M = 256
NB = N // M


def _bn(v):
    return v / jnp.sqrt(1.0 + 1e-5)


def _lrelu(v):
    return jnp.where(v >= 0, v, 0.2 * v)


def _dot(a, b):
    return jax.lax.dot_general(a.astype(jnp.bfloat16), b.astype(jnp.bfloat16),
                               (((1,), (0,)), ((), ())),
                               preferred_element_type=jnp.float32)


def _split3(y):
    y_hi = y.astype(jnp.bfloat16)
    r = y - y_hi.astype(jnp.float32)
    y_mid = r.astype(jnp.bfloat16)
    y_lo = (r - y_mid.astype(jnp.float32)).astype(jnp.bfloat16)
    return y_hi, y_mid, y_lo


def _gather_rows(ohb, ys):
    oh = ohb.astype(jnp.bfloat16)
    dg = lambda a, b: jax.lax.dot_general(
        a, b, (((1,), (0,)), ((), ())), preferred_element_type=jnp.float32)
    y_hi, y_mid, y_lo = ys
    return dg(oh, y_hi) + dg(oh, y_mid) + dg(oh, y_lo)


def _edge_body(f_ref, wd_ref, wc_ref, wb_ref, out_ref, *, second):
    m = pl.program_id(1)
    F = f_ref[0]
    Fblk = f_ref[0, pl.ds(m * M, M), :]
    sq = jnp.sum(F * F, axis=1)
    sqb = jnp.sum(Fblk * Fblk, axis=1)
    D = 2.0 * jax.lax.dot_general(
        Fblk.astype(jnp.bfloat16), F.astype(jnp.bfloat16),
        (((1,), (1,)), ((), ())), preferred_element_type=jnp.float32)
    D = D - sqb[:, None] - sq[None, :]

    iota = jax.lax.broadcasted_iota(jnp.int32, (M, N), 1)
    ys = _split3(F)
    zc = _dot(Fblk, wc_ref[:])

    acc = jnp.full((M, 64), -jnp.inf, jnp.float32)
    for _t in range(K):
        mx = jnp.max(D, axis=1)
        eq = D == mx[:, None]
        ji = jnp.min(jnp.where(eq, iota, N), axis=1)
        ohb = iota == ji[:, None]
        D = jnp.where(ohb, -jnp.inf, D)
        nbr = _gather_rows(ohb, ys)
        p = _dot(nbr - Fblk, wd_ref[:]) + zc
        if second:
            e = _lrelu(_bn(p))
            acc = jnp.maximum(acc, _dot(e, wb_ref[:]))
        else:
            acc = jnp.maximum(acc, p)

    out_ref[0] = _lrelu(_bn(acc))


def _edge_stage(F, wdT, wcT, wbT):
    B, _, C = F.shape
    second = wbT is not None
    ws = [wdT, wcT] + ([wbT] if second else [])
    if second:
        body = functools.partial(_edge_body, second=True)
    else:
        def body(f_ref, wd_ref, wc_ref, out_ref):
            _edge_body(f_ref, wd_ref, wc_ref, None, out_ref, second=False)
    in_specs = [pl.BlockSpec((1, N, C), lambda b, m: (b, 0, 0))]
    in_specs += [pl.BlockSpec(w.shape, lambda b, m: (0, 0)) for w in ws]
    return pl.pallas_call(
        body,
        grid=(B, NB),
        in_specs=in_specs,
        out_specs=pl.BlockSpec((1, M, 64), lambda b, m: (b, m, 0)),
        out_shape=jax.ShapeDtypeStruct((B, N, 64), jnp.float32),
    )(F, *ws)


def _head_body(x1_ref, x2_ref, x3_ref, w6a, w6b, w6c, w8g, w8a, w8b, w8c,
               w9, w10, w11, out_ref):
    x1 = x1_ref[0]
    x2 = x2_ref[0]
    x3 = x3_ref[0]
    gp = _dot(x1, w6a[:]) + _dot(x2, w6b[:]) + _dot(x3, w6c[:])
    g = jnp.max(_lrelu(_bn(gp)), axis=0)
    gv = _dot(g[None, :], w8g[:])
    h = _lrelu(_bn(_dot(x1, w8a[:]) + _dot(x2, w8b[:]) + _dot(x3, w8c[:])
                   + gv))
    h = _lrelu(_bn(_dot(h, w9[:])))
    h = _lrelu(_bn(_dot(h, w10[:])))
    out_ref[0] = _dot(h, w11[:])


def _head(x1, x2, x3, *ws):
    B = x1.shape[0]
    in_specs = [pl.BlockSpec((1, N, 64), lambda b: (b, 0, 0))] * 3
    in_specs += [pl.BlockSpec(w.shape, lambda b: (0, 0)) for w in ws]
    return pl.pallas_call(
        _head_body,
        grid=(B,),
        in_specs=in_specs,
        out_specs=pl.BlockSpec((1, N, 50), lambda b: (b, 0, 0)),
        out_shape=jax.ShapeDtypeStruct((B, N, 50), jnp.float32),
    )(x1, x2, x3, *ws)


def _split_edge_w(W, C, pad=0):
    wd = W[:, :C].T
    wc = W[:, C:].T
    if pad:
        zpad = jnp.zeros((pad, W.shape[0]), W.dtype)
        wd = jnp.concatenate([wd, zpad], axis=0)
        wc = jnp.concatenate([wc, zpad], axis=0)
    return wd, wc


def kernel(x, W1, W2, W3, W4, W5, W6, W8, W9, W10, W11):
    B = x.shape[0]
    xt = jnp.transpose(x, (0, 2, 1))
    xp = jnp.concatenate([xt, jnp.zeros((B, N, 5), xt.dtype)], axis=-1)

    wd1, wc1 = _split_edge_w(W1, 3, pad=5)
    wd2, wc2 = _split_edge_w(W3, 64)
    wd3, wc3 = _split_edge_w(W5, 64)

    x1 = _edge_stage(xp, wd1, wc1, W2.T)
    x2 = _edge_stage(x1, wd2, wc2, W4.T)
    x3 = _edge_stage(x2, wd3, wc3, None)

    outT = _head(x1, x2, x3,
                 W6[:, :64].T, W6[:, 64:128].T, W6[:, 128:].T,
                 W8[:, :1024].T, W8[:, 1024:1088].T, W8[:, 1088:1152].T,
                 W8[:, 1152:].T, W9.T, W10.T, W11.T)
    return jnp.transpose(outT, (0, 2, 1))


# speedup vs baseline: 6.3375x; 1.0493x over previous
import functools

import jax
import jax.numpy as jnp
from jax.experimental import pallas as pl
from jax.experimental.pallas import tpu as pltpu

N = 2048
K = 20
M = 256
NB = N // M


def _bn(v):
    return v / jnp.sqrt(1.0 + 1e-5)


def _lrelu(v):
    return jnp.where(v >= 0, v, 0.2 * v)


def _dot(a, b):
    return jax.lax.dot_general(a.astype(jnp.bfloat16), b.astype(jnp.bfloat16),
                               (((1,), (0,)), ((), ())),
                               preferred_element_type=jnp.float32)


def _split3(y):
    y_hi = y.astype(jnp.bfloat16)
    r = y - y_hi.astype(jnp.float32)
    y_mid = r.astype(jnp.bfloat16)
    y_lo = (r - y_mid.astype(jnp.float32)).astype(jnp.bfloat16)
    return y_hi, y_mid, y_lo


def _gather_rows(ohb, cat3, C):
    g3 = jax.lax.dot_general(ohb.astype(jnp.bfloat16), cat3,
                             (((1,), (0,)), ((), ())),
                             preferred_element_type=jnp.float32)
    return g3[:, :C] + g3[:, C:2 * C] + g3[:, 2 * C:]


def _edge_body(f_ref, wd_ref, wc_ref, wb_ref, out_ref, *, second):
    m = pl.program_id(1)
    F = f_ref[0]
    Fblk = f_ref[0, pl.ds(m * M, M), :]
    sq = jnp.sum(F * F, axis=1)
    sqb = jnp.sum(Fblk * Fblk, axis=1)
    D = 2.0 * jax.lax.dot_general(
        Fblk.astype(jnp.bfloat16), F.astype(jnp.bfloat16),
        (((1,), (1,)), ((), ())), preferred_element_type=jnp.float32)
    D = D - sqb[:, None] - sq[None, :]

    iota = jax.lax.broadcasted_iota(jnp.int32, (M, N), 1)
    C = F.shape[1]
    cat3 = jnp.concatenate(_split3(F), axis=1)
    zc = _dot(Fblk, wc_ref[:])

    acc = jnp.full((M, 64), -jnp.inf, jnp.float32)
    for _t in range(K):
        ji = jnp.argmax(D, axis=1)
        ohb = iota == ji[:, None]
        D = jnp.where(ohb, -jnp.inf, D)
        nbr = _gather_rows(ohb, cat3, C)
        p = _dot(nbr - Fblk, wd_ref[:]) + zc
        if second:
            e = _lrelu(_bn(p))
            acc = jnp.maximum(acc, _dot(e, wb_ref[:]))
        else:
            acc = jnp.maximum(acc, p)

    out_ref[0] = _lrelu(_bn(acc))


def _edge_stage(F, wdT, wcT, wbT):
    B, _, C = F.shape
    second = wbT is not None
    ws = [wdT, wcT] + ([wbT] if second else [])
    if second:
        body = functools.partial(_edge_body, second=True)
    else:
        def body(f_ref, wd_ref, wc_ref, out_ref):
            _edge_body(f_ref, wd_ref, wc_ref, None, out_ref, second=False)
    in_specs = [pl.BlockSpec((1, N, C), lambda b, m: (b, 0, 0))]
    in_specs += [pl.BlockSpec(w.shape, lambda b, m: (0, 0)) for w in ws]
    return pl.pallas_call(
        body,
        grid=(B, NB),
        in_specs=in_specs,
        out_specs=pl.BlockSpec((1, M, 64), lambda b, m: (b, m, 0)),
        out_shape=jax.ShapeDtypeStruct((B, N, 64), jnp.float32),
    )(F, *ws)


def _head_body(x1_ref, x2_ref, x3_ref, w6a, w6b, w6c, w8g, w8a, w8b, w8c,
               w9, w10, w11, out_ref):
    x1 = x1_ref[0]
    x2 = x2_ref[0]
    x3 = x3_ref[0]
    gp = _dot(x1, w6a[:]) + _dot(x2, w6b[:]) + _dot(x3, w6c[:])
    g = jnp.max(_lrelu(_bn(gp)), axis=0)
    gv = _dot(g[None, :], w8g[:])
    h = _lrelu(_bn(_dot(x1, w8a[:]) + _dot(x2, w8b[:]) + _dot(x3, w8c[:])
                   + gv))
    h = _lrelu(_bn(_dot(h, w9[:])))
    h = _lrelu(_bn(_dot(h, w10[:])))
    out_ref[0] = _dot(h, w11[:])


def _head(x1, x2, x3, *ws):
    B = x1.shape[0]
    in_specs = [pl.BlockSpec((1, N, 64), lambda b: (b, 0, 0))] * 3
    in_specs += [pl.BlockSpec(w.shape, lambda b: (0, 0)) for w in ws]
    return pl.pallas_call(
        _head_body,
        grid=(B,),
        in_specs=in_specs,
        out_specs=pl.BlockSpec((1, N, 50), lambda b: (b, 0, 0)),
        out_shape=jax.ShapeDtypeStruct((B, N, 50), jnp.float32),
    )(x1, x2, x3, *ws)


def _split_edge_w(W, C, pad=0):
    wd = W[:, :C].T
    wc = W[:, C:].T
    if pad:
        zpad = jnp.zeros((pad, W.shape[0]), W.dtype)
        wd = jnp.concatenate([wd, zpad], axis=0)
        wc = jnp.concatenate([wc, zpad], axis=0)
    return wd, wc


def kernel(x, W1, W2, W3, W4, W5, W6, W8, W9, W10, W11):
    B = x.shape[0]
    xt = jnp.transpose(x, (0, 2, 1))
    xp = jnp.concatenate([xt, jnp.zeros((B, N, 5), xt.dtype)], axis=-1)

    wd1, wc1 = _split_edge_w(W1, 3, pad=5)
    wd2, wc2 = _split_edge_w(W3, 64)
    wd3, wc3 = _split_edge_w(W5, 64)

    x1 = _edge_stage(xp, wd1, wc1, W2.T)
    x2 = _edge_stage(x1, wd2, wc2, W4.T)
    x3 = _edge_stage(x2, wd3, wc3, None)

    outT = _head(x1, x2, x3,
                 W6[:, :64].T, W6[:, 64:128].T, W6[:, 128:].T,
                 W8[:, :1024].T, W8[:, 1024:1088].T, W8[:, 1088:1152].T,
                 W8[:, 1152:].T, W9.T, W10.T, W11.T)
    return jnp.transpose(outT, (0, 2, 1))


# speedup vs baseline: 9.0291x; 1.4247x over previous
import functools

import jax
import jax.numpy as jnp
from jax.experimental import pallas as pl
from jax.experimental.pallas import tpu as pltpu

N = 2048
K = 20
M = 512
NB = N // M


def _bn(v):
    return v / jnp.sqrt(1.0 + 1e-5)


def _lrelu(v):
    return jnp.where(v >= 0, v, 0.2 * v)


def _dot(a, b):
    return jax.lax.dot_general(a.astype(jnp.bfloat16), b.astype(jnp.bfloat16),
                               (((1,), (0,)), ((), ())),
                               preferred_element_type=jnp.float32)


def _split3(y):
    y_hi = y.astype(jnp.bfloat16)
    r = y - y_hi.astype(jnp.float32)
    y_mid = r.astype(jnp.bfloat16)
    y_lo = (r - y_mid.astype(jnp.float32)).astype(jnp.bfloat16)
    return y_hi, y_mid, y_lo


def _gather_rows(ohb, cat3, C):
    g3 = jax.lax.dot_general(ohb.astype(jnp.bfloat16), cat3,
                             (((1,), (0,)), ((), ())),
                             preferred_element_type=jnp.float32)
    return g3[:, :C] + g3[:, C:2 * C] + g3[:, 2 * C:]


def _edge_body(f_ref, wd_ref, wc_ref, wb_ref, out_ref, *, second):
    m = pl.program_id(1)
    F = f_ref[0]
    Fblk = f_ref[0, pl.ds(m * M, M), :]
    sq = jnp.sum(F * F, axis=1)
    sqb = jnp.sum(Fblk * Fblk, axis=1)
    D = 2.0 * jax.lax.dot_general(
        Fblk.astype(jnp.bfloat16), F.astype(jnp.bfloat16),
        (((1,), (1,)), ((), ())), preferred_element_type=jnp.float32)
    D = D - sqb[:, None] - sq[None, :]

    iota = jax.lax.broadcasted_iota(jnp.int32, (M, N), 1)
    C = F.shape[1]
    cat3 = jnp.concatenate(_split3(F), axis=1)
    zc = _dot(Fblk, wc_ref[:])

    acc = jnp.full((M, 64), -jnp.inf, jnp.float32)
    for _t in range(K):
        ji = jnp.argmax(D, axis=1)
        ohb = iota == ji[:, None]
        D = jnp.where(ohb, -jnp.inf, D)
        nbr = _gather_rows(ohb, cat3, C)
        p = _dot(nbr - Fblk, wd_ref[:]) + zc
        if second:
            e = _lrelu(_bn(p))
            acc = jnp.maximum(acc, _dot(e, wb_ref[:]))
        else:
            acc = jnp.maximum(acc, p)

    out_ref[0] = _lrelu(_bn(acc))


def _edge_stage(F, wdT, wcT, wbT):
    B, _, C = F.shape
    second = wbT is not None
    ws = [wdT, wcT] + ([wbT] if second else [])
    if second:
        body = functools.partial(_edge_body, second=True)
    else:
        def body(f_ref, wd_ref, wc_ref, out_ref):
            _edge_body(f_ref, wd_ref, wc_ref, None, out_ref, second=False)
    in_specs = [pl.BlockSpec((1, N, C), lambda b, m: (b, 0, 0))]
    in_specs += [pl.BlockSpec(w.shape, lambda b, m: (0, 0)) for w in ws]
    return pl.pallas_call(
        body,
        grid=(B, NB),
        in_specs=in_specs,
        out_specs=pl.BlockSpec((1, M, 64), lambda b, m: (b, m, 0)),
        out_shape=jax.ShapeDtypeStruct((B, N, 64), jnp.float32),
    )(F, *ws)


def _head_body(x1_ref, x2_ref, x3_ref, w6a, w6b, w6c, w8g, w8a, w8b, w8c,
               w9, w10, w11, out_ref):
    x1 = x1_ref[0]
    x2 = x2_ref[0]
    x3 = x3_ref[0]
    gp = _dot(x1, w6a[:]) + _dot(x2, w6b[:]) + _dot(x3, w6c[:])
    g = jnp.max(_lrelu(_bn(gp)), axis=0)
    gv = _dot(g[None, :], w8g[:])
    h = _lrelu(_bn(_dot(x1, w8a[:]) + _dot(x2, w8b[:]) + _dot(x3, w8c[:])
                   + gv))
    h = _lrelu(_bn(_dot(h, w9[:])))
    h = _lrelu(_bn(_dot(h, w10[:])))
    out_ref[0] = _dot(h, w11[:])


def _head(x1, x2, x3, *ws):
    B = x1.shape[0]
    in_specs = [pl.BlockSpec((1, N, 64), lambda b: (b, 0, 0))] * 3
    in_specs += [pl.BlockSpec(w.shape, lambda b: (0, 0)) for w in ws]
    return pl.pallas_call(
        _head_body,
        grid=(B,),
        in_specs=in_specs,
        out_specs=pl.BlockSpec((1, N, 50), lambda b: (b, 0, 0)),
        out_shape=jax.ShapeDtypeStruct((B, N, 50), jnp.float32),
    )(x1, x2, x3, *ws)


def _split_edge_w(W, C, pad=0):
    wd = W[:, :C].T
    wc = W[:, C:].T
    if pad:
        zpad = jnp.zeros((pad, W.shape[0]), W.dtype)
        wd = jnp.concatenate([wd, zpad], axis=0)
        wc = jnp.concatenate([wc, zpad], axis=0)
    return wd, wc


def kernel(x, W1, W2, W3, W4, W5, W6, W8, W9, W10, W11):
    B = x.shape[0]
    xt = jnp.transpose(x, (0, 2, 1))
    xp = jnp.concatenate([xt, jnp.zeros((B, N, 5), xt.dtype)], axis=-1)

    wd1, wc1 = _split_edge_w(W1, 3, pad=5)
    wd2, wc2 = _split_edge_w(W3, 64)
    wd3, wc3 = _split_edge_w(W5, 64)

    x1 = _edge_stage(xp, wd1, wc1, W2.T)
    x2 = _edge_stage(x1, wd2, wc2, W4.T)
    x3 = _edge_stage(x2, wd3, wc3, None)

    outT = _head(x1, x2, x3,
                 W6[:, :64].T, W6[:, 64:128].T, W6[:, 128:].T,
                 W8[:, :1024].T, W8[:, 1024:1088].T, W8[:, 1088:1152].T,
                 W8[:, 1152:].T, W9.T, W10.T, W11.T)
    return jnp.transpose(outT, (0, 2, 1))


# speedup vs baseline: 9.1642x; 1.0150x over previous
import functools

import jax
import jax.numpy as jnp
from jax.experimental import pallas as pl
from jax.experimental.pallas import tpu as pltpu

N = 2048
K = 20
M = 1024
NB = N // M


def _bn(v):
    return v / jnp.sqrt(1.0 + 1e-5)


def _lrelu(v):
    return jnp.where(v >= 0, v, 0.2 * v)


def _dot(a, b):
    return jax.lax.dot_general(a.astype(jnp.bfloat16), b.astype(jnp.bfloat16),
                               (((1,), (0,)), ((), ())),
                               preferred_element_type=jnp.float32)


def _split3(y):
    y_hi = y.astype(jnp.bfloat16)
    r = y - y_hi.astype(jnp.float32)
    y_mid = r.astype(jnp.bfloat16)
    y_lo = (r - y_mid.astype(jnp.float32)).astype(jnp.bfloat16)
    return y_hi, y_mid, y_lo


def _gather_rows(ohb, cat3, C):
    g3 = jax.lax.dot_general(ohb.astype(jnp.bfloat16), cat3,
                             (((1,), (0,)), ((), ())),
                             preferred_element_type=jnp.float32)
    return g3[:, :C] + g3[:, C:2 * C] + g3[:, 2 * C:]


def _edge_body(f_ref, wd_ref, wc_ref, wb_ref, out_ref, *, second):
    m = pl.program_id(1)
    F = f_ref[0]
    Fblk = f_ref[0, pl.ds(m * M, M), :]
    sq = jnp.sum(F * F, axis=1)
    sqb = jnp.sum(Fblk * Fblk, axis=1)
    D = 2.0 * jax.lax.dot_general(
        Fblk.astype(jnp.bfloat16), F.astype(jnp.bfloat16),
        (((1,), (1,)), ((), ())), preferred_element_type=jnp.float32)
    D = D - sqb[:, None] - sq[None, :]

    iota = jax.lax.broadcasted_iota(jnp.int32, (M, N), 1)
    C = F.shape[1]
    cat3 = jnp.concatenate(_split3(F), axis=1)
    zc = _dot(Fblk, wc_ref[:])

    acc = jnp.full((M, 64), -jnp.inf, jnp.float32)
    for _t in range(K):
        ji = jnp.argmax(D, axis=1)
        ohb = iota == ji[:, None]
        D = jnp.where(ohb, -jnp.inf, D)
        nbr = _gather_rows(ohb, cat3, C)
        p = _dot(nbr - Fblk, wd_ref[:]) + zc
        if second:
            e = _lrelu(_bn(p))
            acc = jnp.maximum(acc, _dot(e, wb_ref[:]))
        else:
            acc = jnp.maximum(acc, p)

    out_ref[0] = _lrelu(_bn(acc))


def _edge_stage(F, wdT, wcT, wbT):
    B, _, C = F.shape
    second = wbT is not None
    ws = [wdT, wcT] + ([wbT] if second else [])
    if second:
        body = functools.partial(_edge_body, second=True)
    else:
        def body(f_ref, wd_ref, wc_ref, out_ref):
            _edge_body(f_ref, wd_ref, wc_ref, None, out_ref, second=False)
    in_specs = [pl.BlockSpec((1, N, C), lambda b, m: (b, 0, 0))]
    in_specs += [pl.BlockSpec(w.shape, lambda b, m: (0, 0)) for w in ws]
    return pl.pallas_call(
        body,
        grid=(B, NB),
        in_specs=in_specs,
        out_specs=pl.BlockSpec((1, M, 64), lambda b, m: (b, m, 0)),
        out_shape=jax.ShapeDtypeStruct((B, N, 64), jnp.float32),
    )(F, *ws)


def _head_body(x1_ref, x2_ref, x3_ref, w6a, w6b, w6c, w8g, w8a, w8b, w8c,
               w9, w10, w11, out_ref):
    x1 = x1_ref[0]
    x2 = x2_ref[0]
    x3 = x3_ref[0]
    gp = _dot(x1, w6a[:]) + _dot(x2, w6b[:]) + _dot(x3, w6c[:])
    g = jnp.max(_lrelu(_bn(gp)), axis=0)
    gv = _dot(g[None, :], w8g[:])
    h = _lrelu(_bn(_dot(x1, w8a[:]) + _dot(x2, w8b[:]) + _dot(x3, w8c[:])
                   + gv))
    h = _lrelu(_bn(_dot(h, w9[:])))
    h = _lrelu(_bn(_dot(h, w10[:])))
    out_ref[0] = _dot(h, w11[:])


def _head(x1, x2, x3, *ws):
    B = x1.shape[0]
    in_specs = [pl.BlockSpec((1, N, 64), lambda b: (b, 0, 0))] * 3
    in_specs += [pl.BlockSpec(w.shape, lambda b: (0, 0)) for w in ws]
    return pl.pallas_call(
        _head_body,
        grid=(B,),
        in_specs=in_specs,
        out_specs=pl.BlockSpec((1, N, 50), lambda b: (b, 0, 0)),
        out_shape=jax.ShapeDtypeStruct((B, N, 50), jnp.float32),
    )(x1, x2, x3, *ws)


def _split_edge_w(W, C, pad=0):
    wd = W[:, :C].T
    wc = W[:, C:].T
    if pad:
        zpad = jnp.zeros((pad, W.shape[0]), W.dtype)
        wd = jnp.concatenate([wd, zpad], axis=0)
        wc = jnp.concatenate([wc, zpad], axis=0)
    return wd, wc


def kernel(x, W1, W2, W3, W4, W5, W6, W8, W9, W10, W11):
    B = x.shape[0]
    xt = jnp.transpose(x, (0, 2, 1))
    xp = jnp.concatenate([xt, jnp.zeros((B, N, 5), xt.dtype)], axis=-1)

    wd1, wc1 = _split_edge_w(W1, 3, pad=5)
    wd2, wc2 = _split_edge_w(W3, 64)
    wd3, wc3 = _split_edge_w(W5, 64)

    x1 = _edge_stage(xp, wd1, wc1, W2.T)
    x2 = _edge_stage(x1, wd2, wc2, W4.T)
    x3 = _edge_stage(x2, wd3, wc3, None)

    outT = _head(x1, x2, x3,
                 W6[:, :64].T, W6[:, 64:128].T, W6[:, 128:].T,
                 W8[:, :1024].T, W8[:, 1024:1088].T, W8[:, 1088:1152].T,
                 W8[:, 1152:].T, W9.T, W10.T, W11.T)
    return jnp.transpose(outT, (0, 2, 1))


# speedup vs baseline: 10.0567x; 1.0974x over previous
import functools

import jax
import jax.numpy as jnp
from jax.experimental import pallas as pl
from jax.experimental.pallas import tpu as pltpu

N = 2048
K = 20
M = 2048
NB = N // M


def _bn(v):
    return v / jnp.sqrt(1.0 + 1e-5)


def _lrelu(v):
    return jnp.where(v >= 0, v, 0.2 * v)


def _dot(a, b):
    return jax.lax.dot_general(a.astype(jnp.bfloat16), b.astype(jnp.bfloat16),
                               (((1,), (0,)), ((), ())),
                               preferred_element_type=jnp.float32)


def _split3(y):
    y_hi = y.astype(jnp.bfloat16)
    r = y - y_hi.astype(jnp.float32)
    y_mid = r.astype(jnp.bfloat16)
    y_lo = (r - y_mid.astype(jnp.float32)).astype(jnp.bfloat16)
    return y_hi, y_mid, y_lo


def _gather_rows(ohb, cat3, C):
    g3 = jax.lax.dot_general(ohb.astype(jnp.bfloat16), cat3,
                             (((1,), (0,)), ((), ())),
                             preferred_element_type=jnp.float32)
    return g3[:, :C] + g3[:, C:2 * C] + g3[:, 2 * C:]


def _edge_body(f_ref, wd_ref, wc_ref, wb_ref, out_ref, *, second):
    m = pl.program_id(1)
    F = f_ref[0]
    Fblk = f_ref[0, pl.ds(m * M, M), :]
    sq = jnp.sum(F * F, axis=1)
    sqb = jnp.sum(Fblk * Fblk, axis=1)
    D = 2.0 * jax.lax.dot_general(
        Fblk.astype(jnp.bfloat16), F.astype(jnp.bfloat16),
        (((1,), (1,)), ((), ())), preferred_element_type=jnp.float32)
    D = D - sqb[:, None] - sq[None, :]

    iota = jax.lax.broadcasted_iota(jnp.int32, (M, N), 1)
    C = F.shape[1]
    cat3 = jnp.concatenate(_split3(F), axis=1)
    zc = _dot(Fblk, wc_ref[:])

    acc = jnp.full((M, 64), -jnp.inf, jnp.float32)
    for _t in range(K):
        ji = jnp.argmax(D, axis=1)
        ohb = iota == ji[:, None]
        D = jnp.where(ohb, -jnp.inf, D)
        nbr = _gather_rows(ohb, cat3, C)
        p = _dot(nbr - Fblk, wd_ref[:]) + zc
        if second:
            e = _lrelu(_bn(p))
            acc = jnp.maximum(acc, _dot(e, wb_ref[:]))
        else:
            acc = jnp.maximum(acc, p)

    out_ref[0] = _lrelu(_bn(acc))


def _edge_stage(F, wdT, wcT, wbT):
    B, _, C = F.shape
    second = wbT is not None
    ws = [wdT, wcT] + ([wbT] if second else [])
    if second:
        body = functools.partial(_edge_body, second=True)
    else:
        def body(f_ref, wd_ref, wc_ref, out_ref):
            _edge_body(f_ref, wd_ref, wc_ref, None, out_ref, second=False)
    in_specs = [pl.BlockSpec((1, N, C), lambda b, m: (b, 0, 0))]
    in_specs += [pl.BlockSpec(w.shape, lambda b, m: (0, 0)) for w in ws]
    return pl.pallas_call(
        body,
        grid=(B, NB),
        in_specs=in_specs,
        out_specs=pl.BlockSpec((1, M, 64), lambda b, m: (b, m, 0)),
        out_shape=jax.ShapeDtypeStruct((B, N, 64), jnp.float32),
    )(F, *ws)


def _head_body(x1_ref, x2_ref, x3_ref, w6a, w6b, w6c, w8g, w8a, w8b, w8c,
               w9, w10, w11, out_ref):
    x1 = x1_ref[0]
    x2 = x2_ref[0]
    x3 = x3_ref[0]
    gp = _dot(x1, w6a[:]) + _dot(x2, w6b[:]) + _dot(x3, w6c[:])
    g = jnp.max(_lrelu(_bn(gp)), axis=0)
    gv = _dot(g[None, :], w8g[:])
    h = _lrelu(_bn(_dot(x1, w8a[:]) + _dot(x2, w8b[:]) + _dot(x3, w8c[:])
                   + gv))
    h = _lrelu(_bn(_dot(h, w9[:])))
    h = _lrelu(_bn(_dot(h, w10[:])))
    out_ref[0] = _dot(h, w11[:])


def _head(x1, x2, x3, *ws):
    B = x1.shape[0]
    in_specs = [pl.BlockSpec((1, N, 64), lambda b: (b, 0, 0))] * 3
    in_specs += [pl.BlockSpec(w.shape, lambda b: (0, 0)) for w in ws]
    return pl.pallas_call(
        _head_body,
        grid=(B,),
        in_specs=in_specs,
        out_specs=pl.BlockSpec((1, N, 50), lambda b: (b, 0, 0)),
        out_shape=jax.ShapeDtypeStruct((B, N, 50), jnp.float32),
    )(x1, x2, x3, *ws)


def _split_edge_w(W, C, pad=0):
    wd = W[:, :C].T
    wc = W[:, C:].T
    if pad:
        zpad = jnp.zeros((pad, W.shape[0]), W.dtype)
        wd = jnp.concatenate([wd, zpad], axis=0)
        wc = jnp.concatenate([wc, zpad], axis=0)
    return wd, wc


def kernel(x, W1, W2, W3, W4, W5, W6, W8, W9, W10, W11):
    B = x.shape[0]
    xt = jnp.transpose(x, (0, 2, 1))
    xp = jnp.concatenate([xt, jnp.zeros((B, N, 5), xt.dtype)], axis=-1)

    wd1, wc1 = _split_edge_w(W1, 3, pad=5)
    wd2, wc2 = _split_edge_w(W3, 64)
    wd3, wc3 = _split_edge_w(W5, 64)

    x1 = _edge_stage(xp, wd1, wc1, W2.T)
    x2 = _edge_stage(x1, wd2, wc2, W4.T)
    x3 = _edge_stage(x2, wd3, wc3, None)

    outT = _head(x1, x2, x3,
                 W6[:, :64].T, W6[:, 64:128].T, W6[:, 128:].T,
                 W8[:, :1024].T, W8[:, 1024:1088].T, W8[:, 1088:1152].T,
                 W8[:, 1152:].T, W9.T, W10.T, W11.T)
    return jnp.transpose(outT, (0, 2, 1))


# speedup vs baseline: 10.0642x; 1.0007x over previous
import functools

import jax
import jax.numpy as jnp
from jax.experimental import pallas as pl
from jax.experimental.pallas import tpu as pltpu

N = 2048
K = 20
M = 2048
NB = N // M


def _bn(v):
    return v / jnp.sqrt(1.0 + 1e-5)


def _lrelu(v):
    return jnp.where(v >= 0, v, 0.2 * v)


def _dot(a, b):
    return jax.lax.dot_general(a.astype(jnp.bfloat16), b.astype(jnp.bfloat16),
                               (((1,), (0,)), ((), ())),
                               preferred_element_type=jnp.float32)


def _split3(y):
    y_hi = y.astype(jnp.bfloat16)
    r = y - y_hi.astype(jnp.float32)
    y_mid = r.astype(jnp.bfloat16)
    y_lo = (r - y_mid.astype(jnp.float32)).astype(jnp.bfloat16)
    return y_hi, y_mid, y_lo


def _gather_rows(ohb, cat3, C):
    g3 = jax.lax.dot_general(ohb.astype(jnp.bfloat16), cat3,
                             (((1,), (0,)), ((), ())),
                             preferred_element_type=jnp.float32)
    return g3[:, :C] + g3[:, C:2 * C] + g3[:, 2 * C:]


def _edge_body(f_ref, wd_ref, wc_ref, wb_ref, out_ref, *, second):
    m = pl.program_id(1)
    F = f_ref[0]
    Fblk = f_ref[0, pl.ds(m * M, M), :]
    sq = jnp.sum(F * F, axis=1)
    sqb = jnp.sum(Fblk * Fblk, axis=1)
    D = 2.0 * jax.lax.dot_general(
        Fblk.astype(jnp.bfloat16), F.astype(jnp.bfloat16),
        (((1,), (1,)), ((), ())), preferred_element_type=jnp.float32)
    D = D - sqb[:, None] - sq[None, :]

    iota = jax.lax.broadcasted_iota(jnp.int32, (M, N), 1)
    C = F.shape[1]
    cat3 = jnp.concatenate(_split3(F), axis=1)
    zc = _dot(Fblk, wc_ref[:])

    acc = jnp.full((M, 64), -jnp.inf, jnp.float32)
    ohb = None
    for _t in range(K):
        if _t:
            D = jnp.where(ohb, -jnp.inf, D)
        ji = jnp.argmax(D, axis=1)
        ohb = iota == ji[:, None]
        nbr = _gather_rows(ohb, cat3, C)
        p = _dot(nbr - Fblk, wd_ref[:]) + zc
        if second:
            e = _lrelu(_bn(p))
            acc = jnp.maximum(acc, _dot(e, wb_ref[:]))
        else:
            acc = jnp.maximum(acc, p)

    out_ref[0] = _lrelu(_bn(acc))


def _edge_stage(F, wdT, wcT, wbT):
    B, _, C = F.shape
    second = wbT is not None
    ws = [wdT, wcT] + ([wbT] if second else [])
    if second:
        body = functools.partial(_edge_body, second=True)
    else:
        def body(f_ref, wd_ref, wc_ref, out_ref):
            _edge_body(f_ref, wd_ref, wc_ref, None, out_ref, second=False)
    in_specs = [pl.BlockSpec((1, N, C), lambda b, m: (b, 0, 0))]
    in_specs += [pl.BlockSpec(w.shape, lambda b, m: (0, 0)) for w in ws]
    return pl.pallas_call(
        body,
        grid=(B, NB),
        in_specs=in_specs,
        out_specs=pl.BlockSpec((1, M, 64), lambda b, m: (b, m, 0)),
        out_shape=jax.ShapeDtypeStruct((B, N, 64), jnp.float32),
    )(F, *ws)


def _head_body(x1_ref, x2_ref, x3_ref, w6a, w6b, w6c, w8g, w8a, w8b, w8c,
               w9, w10, w11, out_ref):
    x1 = x1_ref[0]
    x2 = x2_ref[0]
    x3 = x3_ref[0]
    gp = _dot(x1, w6a[:]) + _dot(x2, w6b[:]) + _dot(x3, w6c[:])
    g = jnp.max(_lrelu(_bn(gp)), axis=0)
    gv = _dot(g[None, :], w8g[:])
    h = _lrelu(_bn(_dot(x1, w8a[:]) + _dot(x2, w8b[:]) + _dot(x3, w8c[:])
                   + gv))
    h = _lrelu(_bn(_dot(h, w9[:])))
    h = _lrelu(_bn(_dot(h, w10[:])))
    out_ref[0] = _dot(h, w11[:])


def _head(x1, x2, x3, *ws):
    B = x1.shape[0]
    in_specs = [pl.BlockSpec((1, N, 64), lambda b: (b, 0, 0))] * 3
    in_specs += [pl.BlockSpec(w.shape, lambda b: (0, 0)) for w in ws]
    return pl.pallas_call(
        _head_body,
        grid=(B,),
        in_specs=in_specs,
        out_specs=pl.BlockSpec((1, N, 50), lambda b: (b, 0, 0)),
        out_shape=jax.ShapeDtypeStruct((B, N, 50), jnp.float32),
    )(x1, x2, x3, *ws)


def _split_edge_w(W, C, pad=0):
    wd = W[:, :C].T
    wc = W[:, C:].T
    if pad:
        zpad = jnp.zeros((pad, W.shape[0]), W.dtype)
        wd = jnp.concatenate([wd, zpad], axis=0)
        wc = jnp.concatenate([wc, zpad], axis=0)
    return wd, wc


def kernel(x, W1, W2, W3, W4, W5, W6, W8, W9, W10, W11):
    B = x.shape[0]
    xt = jnp.transpose(x, (0, 2, 1))
    xp = jnp.concatenate([xt, jnp.zeros((B, N, 5), xt.dtype)], axis=-1)

    wd1, wc1 = _split_edge_w(W1, 3, pad=5)
    wd2, wc2 = _split_edge_w(W3, 64)
    wd3, wc3 = _split_edge_w(W5, 64)

    x1 = _edge_stage(xp, wd1, wc1, W2.T)
    x2 = _edge_stage(x1, wd2, wc2, W4.T)
    x3 = _edge_stage(x2, wd3, wc3, None)

    outT = _head(x1, x2, x3,
                 W6[:, :64].T, W6[:, 64:128].T, W6[:, 128:].T,
                 W8[:, :1024].T, W8[:, 1024:1088].T, W8[:, 1088:1152].T,
                 W8[:, 1152:].T, W9.T, W10.T, W11.T)
    return jnp.transpose(outT, (0, 2, 1))


# speedup vs baseline: 11.6933x; 1.1619x over previous
import functools

import jax
import jax.numpy as jnp
from jax import lax
from jax.experimental import pallas as pl
from jax.experimental.pallas import tpu as pltpu
from jax.experimental.pallas import tpu_sc as plsc

N = 2048
K = 20
M = 2048
NB = N // M


def _bn(v):
    return v / jnp.sqrt(1.0 + 1e-5)


def _lrelu(v):
    return jnp.where(v >= 0, v, 0.2 * v)


def _dot(a, b):
    return jax.lax.dot_general(a.astype(jnp.bfloat16), b.astype(jnp.bfloat16),
                               (((1,), (0,)), ((), ())),
                               preferred_element_type=jnp.float32)


def _split3(y):
    y_hi = y.astype(jnp.bfloat16)
    r = y - y_hi.astype(jnp.float32)
    y_mid = r.astype(jnp.bfloat16)
    y_lo = (r - y_mid.astype(jnp.float32)).astype(jnp.bfloat16)
    return y_hi, y_mid, y_lo


def _gather_rows(ohb, cat3, C):
    g3 = jax.lax.dot_general(ohb.astype(jnp.bfloat16), cat3,
                             (((1,), (0,)), ((), ())),
                             preferred_element_type=jnp.float32)
    return g3[:, :C] + g3[:, C:2 * C] + g3[:, 2 * C:]


def _edge_body(f_ref, wd_ref, wc_ref, wb_ref, out_ref, *, second):
    m = pl.program_id(1)
    F = f_ref[0]
    Fblk = f_ref[0, pl.ds(m * M, M), :]
    sq = jnp.sum(F * F, axis=1)
    sqb = jnp.sum(Fblk * Fblk, axis=1)
    D = 2.0 * jax.lax.dot_general(
        Fblk.astype(jnp.bfloat16), F.astype(jnp.bfloat16),
        (((1,), (1,)), ((), ())), preferred_element_type=jnp.float32)
    D = D - sqb[:, None] - sq[None, :]

    iota = jax.lax.broadcasted_iota(jnp.int32, (M, N), 1)
    C = F.shape[1]
    cat3 = jnp.concatenate(_split3(F), axis=1)
    zc = _dot(Fblk, wc_ref[:])

    acc = jnp.full((M, 64), -jnp.inf, jnp.float32)
    ohb = None
    for _t in range(K):
        if _t:
            D = jnp.where(ohb, -jnp.inf, D)
        ji = jnp.argmax(D, axis=1)
        ohb = iota == ji[:, None]
        nbr = _gather_rows(ohb, cat3, C)
        p = _dot(nbr - Fblk, wd_ref[:]) + zc
        if second:
            e = _lrelu(_bn(p))
            acc = jnp.maximum(acc, _dot(e, wb_ref[:]))
        else:
            acc = jnp.maximum(acc, p)

    out_ref[0] = _lrelu(_bn(acc))


def _edge_stage(F, wdT, wcT, wbT):
    B, _, C = F.shape
    second = wbT is not None
    ws = [wdT, wcT] + ([wbT] if second else [])
    if second:
        body = functools.partial(_edge_body, second=True)
    else:
        def body(f_ref, wd_ref, wc_ref, out_ref):
            _edge_body(f_ref, wd_ref, wc_ref, None, out_ref, second=False)
    in_specs = [pl.BlockSpec((1, N, C), lambda b, m: (b, 0, 0))]
    in_specs += [pl.BlockSpec(w.shape, lambda b, m: (0, 0)) for w in ws]
    return pl.pallas_call(
        body,
        grid=(B, NB),
        in_specs=in_specs,
        out_specs=pl.BlockSpec((1, M, 64), lambda b, m: (b, m, 0)),
        out_shape=jax.ShapeDtypeStruct((B, N, 64), jnp.float32),
    )(F, *ws)


def _edge_idx_body(f_ref, idx_ref):
    b = pl.program_id(0)
    F = f_ref[0]
    sq = jnp.sum(F * F, axis=1)
    D = 2.0 * jax.lax.dot_general(
        F.astype(jnp.bfloat16), F.astype(jnp.bfloat16),
        (((1,), (1,)), ((), ())), preferred_element_type=jnp.float32)
    D = D - sq[:, None] - sq[None, :]
    iota = jax.lax.broadcasted_iota(jnp.int32, (N, N), 1)
    ohb = None
    jis = []
    for _t in range(K):
        if _t:
            D = jnp.where(ohb, -jnp.inf, D)
        ji = jnp.argmax(D, axis=1)
        ohb = iota == ji[:, None]
        jis.append(ji + b * N)
    idx_ref[0] = jnp.stack(jis, axis=0)


def _edge_idx(F):
    B, _, C = F.shape
    return pl.pallas_call(
        _edge_idx_body,
        grid=(B,),
        in_specs=[pl.BlockSpec((1, N, C), lambda b: (b, 0, 0))],
        out_specs=pl.BlockSpec((1, K, N), lambda b: (b, 0, 0)),
        out_shape=jax.ShapeDtypeStruct((B, K, N), jnp.int32),
    )(F)


_CH = 128


def _sc_gather(table_pad, idx):
    R = table_pad.shape[0]
    TOT = idx.size
    info = plsc.get_sparse_core_info()
    nw = info.num_cores * info.num_subcores
    per_w = TOT // nw
    nch = per_w // _CH
    mesh = plsc.VectorSubcoreMesh(core_axis_name="c", subcore_axis_name="s")

    @functools.partial(
        pl.kernel, mesh=mesh,
        out_type=jax.ShapeDtypeStruct((TOT, 128), jnp.float32),
        scratch_types=[
            pltpu.VMEM((nch, _CH), jnp.int32),
            pltpu.VMEM((_CH, 128), jnp.float32),
            pltpu.VMEM((_CH, 128), jnp.float32),
            pltpu.SemaphoreType.DMA,
            pltpu.SemaphoreType.DMA,
        ],
    )
    def k(table_hbm, idx_hbm, out_hbm, idx_v, buf0, buf1, sem0, sem1):
        wid = lax.axis_index("s") * info.num_cores + lax.axis_index("c")
        base = wid * per_w
        pltpu.sync_copy(idx_hbm.at[pl.ds(wid * nch, nch)], idx_v)
        bufs = (buf0, buf1)
        sems = (sem0, sem1)
        cps = [None, None]
        for j in range(nch):
            cps[j % 2] = pltpu.async_copy(
                table_hbm.at[idx_v.at[j]], bufs[j % 2], sems[j % 2])
            if j > 0:
                cps[(j - 1) % 2].wait()
                pltpu.sync_copy(bufs[(j - 1) % 2],
                                out_hbm.at[pl.ds(base + (j - 1) * _CH, _CH)])
        cps[(nch - 1) % 2].wait()
        pltpu.sync_copy(bufs[(nch - 1) % 2],
                        out_hbm.at[pl.ds(base + (nch - 1) * _CH, _CH)])

    return k(table_pad, idx.reshape(TOT // _CH, _CH))


def _edge_conv_body(f_ref, nbr_ref, wd_ref, wc_ref, wb_ref, out_ref, *,
                    second, C, NH):
    Fh = f_ref[0]
    nbc = nbr_ref[0][:, :, :C]
    diff = nbc - Fh[None]
    zc = _dot(Fh, wc_ref[:])
    p = _dot(diff.reshape(K * NH, C), wd_ref[:]).reshape(K, NH, 64)
    p = p + zc[None]
    if second:
        e = _lrelu(_bn(p))
        q = _dot(e.reshape(K * NH, 64), wb_ref[:]).reshape(K, NH, 64)
        acc = jnp.max(q, axis=0)
    else:
        acc = jnp.max(p, axis=0)
    out_ref[0] = _lrelu(_bn(acc))


def _edge_stage_sc(F, wdT, wcT, wbT):
    B, _, C = F.shape
    idx = _edge_idx(F)
    tpad = jnp.concatenate(
        [F.reshape(B * N, C), jnp.zeros((B * N, 128 - C), F.dtype)], axis=1)
    nbr = _sc_gather(tpad, idx).reshape(B, K, N, 128)
    NHALF = N // 2
    second = wbT is not None
    ws = [wdT, wcT] + ([wbT] if second else [])
    if second:
        body = functools.partial(_edge_conv_body, second=True, C=C, NH=NHALF)
    else:
        def body(f_ref, nbr_ref, wd_ref, wc_ref, out_ref):
            _edge_conv_body(f_ref, nbr_ref, wd_ref, wc_ref, None, out_ref,
                            second=False, C=C, NH=NHALF)
    in_specs = [pl.BlockSpec((1, NHALF, C), lambda b, h: (b, h, 0)),
                pl.BlockSpec((1, K, NHALF, 128), lambda b, h: (b, 0, h, 0))]
    in_specs += [pl.BlockSpec(w.shape, lambda b, h: (0, 0)) for w in ws]
    return pl.pallas_call(
        body,
        grid=(B, 2),
        in_specs=in_specs,
        out_specs=pl.BlockSpec((1, NHALF, 64), lambda b, h: (b, h, 0)),
        out_shape=jax.ShapeDtypeStruct((B, N, 64), jnp.float32),
    )(F, nbr, *ws)


def _head_body(x1_ref, x2_ref, x3_ref, w6a, w6b, w6c, w8g, w8a, w8b, w8c,
               w9, w10, w11, out_ref):
    x1 = x1_ref[0]
    x2 = x2_ref[0]
    x3 = x3_ref[0]
    gp = _dot(x1, w6a[:]) + _dot(x2, w6b[:]) + _dot(x3, w6c[:])
    g = jnp.max(_lrelu(_bn(gp)), axis=0)
    gv = _dot(g[None, :], w8g[:])
    h = _lrelu(_bn(_dot(x1, w8a[:]) + _dot(x2, w8b[:]) + _dot(x3, w8c[:])
                   + gv))
    h = _lrelu(_bn(_dot(h, w9[:])))
    h = _lrelu(_bn(_dot(h, w10[:])))
    out_ref[0] = _dot(h, w11[:])


def _head(x1, x2, x3, *ws):
    B = x1.shape[0]
    in_specs = [pl.BlockSpec((1, N, 64), lambda b: (b, 0, 0))] * 3
    in_specs += [pl.BlockSpec(w.shape, lambda b: (0, 0)) for w in ws]
    return pl.pallas_call(
        _head_body,
        grid=(B,),
        in_specs=in_specs,
        out_specs=pl.BlockSpec((1, N, 50), lambda b: (b, 0, 0)),
        out_shape=jax.ShapeDtypeStruct((B, N, 50), jnp.float32),
    )(x1, x2, x3, *ws)


def _split_edge_w(W, C, pad=0):
    wd = W[:, :C].T
    wc = W[:, C:].T
    if pad:
        zpad = jnp.zeros((pad, W.shape[0]), W.dtype)
        wd = jnp.concatenate([wd, zpad], axis=0)
        wc = jnp.concatenate([wc, zpad], axis=0)
    return wd, wc


def kernel(x, W1, W2, W3, W4, W5, W6, W8, W9, W10, W11):
    B = x.shape[0]
    xt = jnp.transpose(x, (0, 2, 1))
    xp = jnp.concatenate([xt, jnp.zeros((B, N, 5), xt.dtype)], axis=-1)

    wd1, wc1 = _split_edge_w(W1, 3, pad=5)
    wd2, wc2 = _split_edge_w(W3, 64)
    wd3, wc3 = _split_edge_w(W5, 64)

    x1 = _edge_stage(xp, wd1, wc1, W2.T)
    x2 = _edge_stage_sc(x1, wd2, wc2, W4.T)
    x3 = _edge_stage_sc(x2, wd3, wc3, None)

    outT = _head(x1, x2, x3,
                 W6[:, :64].T, W6[:, 64:128].T, W6[:, 128:].T,
                 W8[:, :1024].T, W8[:, 1024:1088].T, W8[:, 1088:1152].T,
                 W8[:, 1152:].T, W9.T, W10.T, W11.T)
    return jnp.transpose(outT, (0, 2, 1))


# speedup vs baseline: 12.9599x; 1.1083x over previous
import functools

import jax
import jax.numpy as jnp
from jax import lax
from jax.experimental import pallas as pl
from jax.experimental.pallas import tpu as pltpu
from jax.experimental.pallas import tpu_sc as plsc

N = 2048
K = 20
M = 2048
NB = N // M


def _bn(v):
    return v / jnp.sqrt(1.0 + 1e-5)


def _lrelu(v):
    return jnp.where(v >= 0, v, 0.2 * v)


def _dot(a, b):
    return jax.lax.dot_general(a.astype(jnp.bfloat16), b.astype(jnp.bfloat16),
                               (((1,), (0,)), ((), ())),
                               preferred_element_type=jnp.float32)


def _split3(y):
    y_hi = y.astype(jnp.bfloat16)
    r = y - y_hi.astype(jnp.float32)
    y_mid = r.astype(jnp.bfloat16)
    y_lo = (r - y_mid.astype(jnp.float32)).astype(jnp.bfloat16)
    return y_hi, y_mid, y_lo


def _gather_rows(ohb, cat3, C):
    g3 = jax.lax.dot_general(ohb.astype(jnp.bfloat16), cat3,
                             (((1,), (0,)), ((), ())),
                             preferred_element_type=jnp.float32)
    return g3[:, :C] + g3[:, C:2 * C] + g3[:, 2 * C:]


def _edge_body(f_ref, wd_ref, wc_ref, wb_ref, out_ref, *, second):
    m = pl.program_id(1)
    F = f_ref[0]
    Fblk = f_ref[0, pl.ds(m * M, M), :]
    sq = jnp.sum(F * F, axis=1)
    sqb = jnp.sum(Fblk * Fblk, axis=1)
    D = 2.0 * jax.lax.dot_general(
        Fblk.astype(jnp.bfloat16), F.astype(jnp.bfloat16),
        (((1,), (1,)), ((), ())), preferred_element_type=jnp.float32)
    D = D - sqb[:, None] - sq[None, :]

    iota = jax.lax.broadcasted_iota(jnp.int32, (M, N), 1)
    C = F.shape[1]
    cat3 = jnp.concatenate(_split3(F), axis=1)
    zc = _dot(Fblk, wc_ref[:])

    acc = jnp.full((M, 64), -jnp.inf, jnp.float32)
    ohb = None
    for _t in range(K):
        if _t:
            D = jnp.where(ohb, -jnp.inf, D)
        ji = jnp.argmax(D, axis=1)
        ohb = iota == ji[:, None]
        nbr = _gather_rows(ohb, cat3, C)
        p = _dot(nbr - Fblk, wd_ref[:]) + zc
        if second:
            e = _lrelu(_bn(p))
            acc = jnp.maximum(acc, _dot(e, wb_ref[:]))
        else:
            acc = jnp.maximum(acc, p)

    out_ref[0] = _lrelu(_bn(acc))


def _edge_stage(F, wdT, wcT, wbT):
    B, _, C = F.shape
    second = wbT is not None
    ws = [wdT, wcT] + ([wbT] if second else [])
    if second:
        body = functools.partial(_edge_body, second=True)
    else:
        def body(f_ref, wd_ref, wc_ref, out_ref):
            _edge_body(f_ref, wd_ref, wc_ref, None, out_ref, second=False)
    in_specs = [pl.BlockSpec((1, N, C), lambda b, m: (b, 0, 0))]
    in_specs += [pl.BlockSpec(w.shape, lambda b, m: (0, 0)) for w in ws]
    return pl.pallas_call(
        body,
        grid=(B, NB),
        in_specs=in_specs,
        out_specs=pl.BlockSpec((1, M, 64), lambda b, m: (b, m, 0)),
        out_shape=jax.ShapeDtypeStruct((B, N, 64), jnp.float32),
    )(F, *ws)


def _edge_idx_body(f_ref, idx_ref):
    b = pl.program_id(0)
    F = f_ref[0]
    sq = jnp.sum(F * F, axis=1)
    D = 2.0 * jax.lax.dot_general(
        F.astype(jnp.bfloat16), F.astype(jnp.bfloat16),
        (((1,), (1,)), ((), ())), preferred_element_type=jnp.float32)
    D = D - sq[:, None] - sq[None, :]
    iota = jax.lax.broadcasted_iota(jnp.int32, (N, N), 1)
    ohb = None
    jis = []
    for _t in range(K):
        if _t:
            D = jnp.where(ohb, -jnp.inf, D)
        ji = jnp.argmax(D, axis=1)
        ohb = iota == ji[:, None]
        jis.append(ji + b * N)
    idx_ref[0] = jnp.stack(jis, axis=0)


def _edge_idx(F):
    B, _, C = F.shape
    return pl.pallas_call(
        _edge_idx_body,
        grid=(B,),
        in_specs=[pl.BlockSpec((1, N, C), lambda b: (b, 0, 0))],
        out_specs=pl.BlockSpec((1, K, N), lambda b: (b, 0, 0)),
        out_shape=jax.ShapeDtypeStruct((B, K, N), jnp.int32),
    )(F)


_CH = 128


def _sc_gather(table_pad, idx):
    R = table_pad.shape[0]
    TOT = idx.size
    info = plsc.get_sparse_core_info()
    nw = info.num_cores * info.num_subcores
    per_w = TOT // nw
    nch = per_w // _CH
    mesh = plsc.VectorSubcoreMesh(core_axis_name="c", subcore_axis_name="s")

    @functools.partial(
        pl.kernel, mesh=mesh,
        out_type=jax.ShapeDtypeStruct((TOT, 128), jnp.float32),
        scratch_types=[
            pltpu.VMEM((nch, _CH), jnp.int32),
            pltpu.VMEM((_CH, 128), jnp.float32),
            pltpu.VMEM((_CH, 128), jnp.float32),
            pltpu.SemaphoreType.DMA,
            pltpu.SemaphoreType.DMA,
        ],
    )
    def k(table_hbm, idx_hbm, out_hbm, idx_v, buf0, buf1, sem0, sem1):
        wid = lax.axis_index("s") * info.num_cores + lax.axis_index("c")
        base = wid * per_w
        pltpu.sync_copy(idx_hbm.at[pl.ds(wid * nch, nch)], idx_v)
        bufs = (buf0, buf1)
        sems = (sem0, sem1)
        cps = [None, None]
        for j in range(nch):
            cps[j % 2] = pltpu.async_copy(
                table_hbm.at[idx_v.at[j]], bufs[j % 2], sems[j % 2])
            if j > 0:
                cps[(j - 1) % 2].wait()
                pltpu.sync_copy(bufs[(j - 1) % 2],
                                out_hbm.at[pl.ds(base + (j - 1) * _CH, _CH)])
        cps[(nch - 1) % 2].wait()
        pltpu.sync_copy(bufs[(nch - 1) % 2],
                        out_hbm.at[pl.ds(base + (nch - 1) * _CH, _CH)])

    return k(table_pad, idx.reshape(TOT // _CH, _CH))


def _edge_conv_body(f_ref, nbr_ref, wd_ref, wc_ref, wb_ref, out_ref, *,
                    second, C, NH):
    Fh = f_ref[0]
    nbc = nbr_ref[0][:, :, :C]
    diff = nbc - Fh[None]
    zc = _dot(Fh, wc_ref[:])
    p = _dot(diff.reshape(K * NH, C), wd_ref[:]).reshape(K, NH, 64)
    p = p + zc[None]
    if second:
        e = _lrelu(_bn(p))
        q = _dot(e.reshape(K * NH, 64), wb_ref[:]).reshape(K, NH, 64)
        acc = jnp.max(q, axis=0)
    else:
        acc = jnp.max(p, axis=0)
    out_ref[0] = _lrelu(_bn(acc))


def _edge_stage_sc(F, wdT, wcT, wbT):
    B, _, C = F.shape
    idx = _edge_idx(F)
    tpad = jnp.concatenate(
        [F.reshape(B * N, C), jnp.zeros((B * N, 128 - C), F.dtype)], axis=1)
    nbr = _sc_gather(tpad, idx).reshape(B, K, N, 128)
    NHALF = N // 2
    second = wbT is not None
    ws = [wdT, wcT] + ([wbT] if second else [])
    if second:
        body = functools.partial(_edge_conv_body, second=True, C=C, NH=NHALF)
    else:
        def body(f_ref, nbr_ref, wd_ref, wc_ref, out_ref):
            _edge_conv_body(f_ref, nbr_ref, wd_ref, wc_ref, None, out_ref,
                            second=False, C=C, NH=NHALF)
    in_specs = [pl.BlockSpec((1, NHALF, C), lambda b, h: (b, h, 0)),
                pl.BlockSpec((1, K, NHALF, 128), lambda b, h: (b, 0, h, 0))]
    in_specs += [pl.BlockSpec(w.shape, lambda b, h: (0, 0)) for w in ws]
    return pl.pallas_call(
        body,
        grid=(B, 2),
        in_specs=in_specs,
        out_specs=pl.BlockSpec((1, NHALF, 64), lambda b, h: (b, h, 0)),
        out_shape=jax.ShapeDtypeStruct((B, N, 64), jnp.float32),
    )(F, nbr, *ws)


def _head_body(x1_ref, x2_ref, x3_ref, w6a, w6b, w6c, w8g, w8a, w8b, w8c,
               w9, w10, w11, out_ref):
    x1 = x1_ref[0]
    x2 = x2_ref[0]
    x3 = x3_ref[0]
    gp = _dot(x1, w6a[:]) + _dot(x2, w6b[:]) + _dot(x3, w6c[:])
    g = jnp.max(_lrelu(_bn(gp)), axis=0)
    gv = _dot(g[None, :], w8g[:])
    h = _lrelu(_bn(_dot(x1, w8a[:]) + _dot(x2, w8b[:]) + _dot(x3, w8c[:])
                   + gv))
    h = _lrelu(_bn(_dot(h, w9[:])))
    h = _lrelu(_bn(_dot(h, w10[:])))
    out_ref[0] = _dot(h, w11[:])


def _head(x1, x2, x3, *ws):
    B = x1.shape[0]
    in_specs = [pl.BlockSpec((1, N, 64), lambda b: (b, 0, 0))] * 3
    in_specs += [pl.BlockSpec(w.shape, lambda b: (0, 0)) for w in ws]
    return pl.pallas_call(
        _head_body,
        grid=(B,),
        in_specs=in_specs,
        out_specs=pl.BlockSpec((1, N, 50), lambda b: (b, 0, 0)),
        out_shape=jax.ShapeDtypeStruct((B, N, 50), jnp.float32),
    )(x1, x2, x3, *ws)


def _split_edge_w(W, C, pad=0):
    wd = W[:, :C].T
    wc = W[:, C:].T
    if pad:
        zpad = jnp.zeros((pad, W.shape[0]), W.dtype)
        wd = jnp.concatenate([wd, zpad], axis=0)
        wc = jnp.concatenate([wc, zpad], axis=0)
    return wd, wc


def kernel(x, W1, W2, W3, W4, W5, W6, W8, W9, W10, W11):
    B = x.shape[0]
    xt = jnp.transpose(x, (0, 2, 1))
    xp = jnp.concatenate([xt, jnp.zeros((B, N, 5), xt.dtype)], axis=-1)

    wd1, wc1 = _split_edge_w(W1, 3, pad=5)
    wd2, wc2 = _split_edge_w(W3, 64)
    wd3, wc3 = _split_edge_w(W5, 64)

    x1 = _edge_stage_sc(xp, wd1, wc1, W2.T)
    x2 = _edge_stage_sc(x1, wd2, wc2, W4.T)
    x3 = _edge_stage_sc(x2, wd3, wc3, None)

    outT = _head(x1, x2, x3,
                 W6[:, :64].T, W6[:, 64:128].T, W6[:, 128:].T,
                 W8[:, :1024].T, W8[:, 1024:1088].T, W8[:, 1088:1152].T,
                 W8[:, 1152:].T, W9.T, W10.T, W11.T)
    return jnp.transpose(outT, (0, 2, 1))


# speedup vs baseline: 12.9642x; 1.0003x over previous
import functools

import jax
import jax.numpy as jnp
from jax import lax
from jax.experimental import pallas as pl
from jax.experimental.pallas import tpu as pltpu
from jax.experimental.pallas import tpu_sc as plsc

N = 2048
K = 20


def _bn(v):
    return v / jnp.sqrt(1.0 + 1e-5)


def _lrelu(v):
    return jnp.where(v >= 0, v, 0.2 * v)


def _dot(a, b):
    return jax.lax.dot_general(a.astype(jnp.bfloat16), b.astype(jnp.bfloat16),
                               (((1,), (0,)), ((), ())),
                               preferred_element_type=jnp.float32)


def _edge_idx_body(f_ref, idx_ref):
    b = pl.program_id(0)
    F = f_ref[0]
    sq = jnp.sum(F * F, axis=1)
    D = 2.0 * jax.lax.dot_general(
        F.astype(jnp.bfloat16), F.astype(jnp.bfloat16),
        (((1,), (1,)), ((), ())), preferred_element_type=jnp.float32)
    D = D - sq[:, None] - sq[None, :]
    iota = jax.lax.broadcasted_iota(jnp.int32, (N, N), 1)
    ohb = None
    jis = []
    for _t in range(K):
        if _t:
            D = jnp.where(ohb, -jnp.inf, D)
        ji = jnp.argmax(D, axis=1)
        ohb = iota == ji[:, None]
        jis.append(ji + b * N)
    idx_ref[0] = jnp.stack(jis, axis=0)


def _edge_idx(F):
    B, _, C = F.shape
    return pl.pallas_call(
        _edge_idx_body,
        grid=(B,),
        in_specs=[pl.BlockSpec((1, N, C), lambda b: (b, 0, 0))],
        out_specs=pl.BlockSpec((1, K, N), lambda b: (b, 0, 0)),
        out_shape=jax.ShapeDtypeStruct((B, K, N), jnp.int32),
    )(F)


_CH = 128


def _sc_gather(table_pad, idx):
    R = table_pad.shape[0]
    TOT = idx.size
    info = plsc.get_sparse_core_info()
    nw = info.num_cores * info.num_subcores
    per_w = TOT // nw
    nch = per_w // _CH
    mesh = plsc.VectorSubcoreMesh(core_axis_name="c", subcore_axis_name="s")

    @functools.partial(
        pl.kernel, mesh=mesh,
        out_type=jax.ShapeDtypeStruct((TOT, 128), jnp.float32),
        scratch_types=[
            pltpu.VMEM((nch, _CH), jnp.int32),
            pltpu.VMEM((_CH, 128), jnp.float32),
            pltpu.VMEM((_CH, 128), jnp.float32),
            pltpu.SemaphoreType.DMA,
            pltpu.SemaphoreType.DMA,
        ],
    )
    def k(table_hbm, idx_hbm, out_hbm, idx_v, buf0, buf1, sem0, sem1):
        wid = lax.axis_index("s") * info.num_cores + lax.axis_index("c")
        base = wid * per_w
        pltpu.sync_copy(idx_hbm.at[pl.ds(wid * nch, nch)], idx_v)
        bufs = (buf0, buf1)
        sems = (sem0, sem1)
        cps = [None, None]
        for j in range(nch):
            cps[j % 2] = pltpu.async_copy(
                table_hbm.at[idx_v.at[j]], bufs[j % 2], sems[j % 2])
            if j > 0:
                cps[(j - 1) % 2].wait()
                pltpu.sync_copy(bufs[(j - 1) % 2],
                                out_hbm.at[pl.ds(base + (j - 1) * _CH, _CH)])
        cps[(nch - 1) % 2].wait()
        pltpu.sync_copy(bufs[(nch - 1) % 2],
                        out_hbm.at[pl.ds(base + (nch - 1) * _CH, _CH)])

    return k(table_pad, idx.reshape(TOT // _CH, _CH))


def _edge_conv_body(f_ref, nbr_ref, wd_ref, wc_ref, wb_ref, out_ref, *,
                    second, C, NH):
    Fh = f_ref[0]
    nbc = nbr_ref[0][:, :, :C]
    diff = nbc - Fh[None]
    zc = _dot(Fh, wc_ref[:])
    p = _dot(diff.reshape(K * NH, C), wd_ref[:]).reshape(K, NH, 64)
    p = p + zc[None]
    if second:
        e = _lrelu(_bn(p))
        q = _dot(e.reshape(K * NH, 64), wb_ref[:]).reshape(K, NH, 64)
        acc = jnp.max(q, axis=0)
    else:
        acc = jnp.max(p, axis=0)
    out_ref[0] = _lrelu(_bn(acc))


def _edge_stage_sc(F, wdT, wcT, wbT):
    B, _, C = F.shape
    idx = _edge_idx(F)
    tpad = jnp.concatenate(
        [F.reshape(B * N, C), jnp.zeros((B * N, 128 - C), F.dtype)], axis=1)
    nbr = _sc_gather(tpad, idx).reshape(B, K, N, 128)
    NHALF = N // 2
    second = wbT is not None
    ws = [wdT, wcT] + ([wbT] if second else [])
    if second:
        body = functools.partial(_edge_conv_body, second=True, C=C, NH=NHALF)
    else:
        def body(f_ref, nbr_ref, wd_ref, wc_ref, out_ref):
            _edge_conv_body(f_ref, nbr_ref, wd_ref, wc_ref, None, out_ref,
                            second=False, C=C, NH=NHALF)
    in_specs = [pl.BlockSpec((1, NHALF, C), lambda b, h: (b, h, 0)),
                pl.BlockSpec((1, K, NHALF, 128), lambda b, h: (b, 0, h, 0))]
    in_specs += [pl.BlockSpec(w.shape, lambda b, h: (0, 0)) for w in ws]
    return pl.pallas_call(
        body,
        grid=(B, 2),
        in_specs=in_specs,
        out_specs=pl.BlockSpec((1, NHALF, 64), lambda b, h: (b, h, 0)),
        out_shape=jax.ShapeDtypeStruct((B, N, 64), jnp.float32),
    )(F, nbr, *ws)


def _head_body(x1_ref, x2_ref, x3_ref, w6a, w6b, w6c, w8g, w8a, w8b, w8c,
               w9, w10, w11, out_ref):
    x1 = x1_ref[0]
    x2 = x2_ref[0]
    x3 = x3_ref[0]
    gp = _dot(x1, w6a[:]) + _dot(x2, w6b[:]) + _dot(x3, w6c[:])
    g = jnp.max(_lrelu(_bn(gp)), axis=0)
    gv = _dot(g[None, :], w8g[:])
    h = _lrelu(_bn(_dot(x1, w8a[:]) + _dot(x2, w8b[:]) + _dot(x3, w8c[:])
                   + gv))
    h = _lrelu(_bn(_dot(h, w9[:])))
    h = _lrelu(_bn(_dot(h, w10[:])))
    out_ref[0] = _dot(h, w11[:])


def _head(x1, x2, x3, *ws):
    B = x1.shape[0]
    in_specs = [pl.BlockSpec((1, N, 64), lambda b: (b, 0, 0))] * 3
    in_specs += [pl.BlockSpec(w.shape, lambda b: (0, 0)) for w in ws]
    return pl.pallas_call(
        _head_body,
        grid=(B,),
        in_specs=in_specs,
        out_specs=pl.BlockSpec((1, N, 50), lambda b: (b, 0, 0)),
        out_shape=jax.ShapeDtypeStruct((B, N, 50), jnp.float32),
    )(x1, x2, x3, *ws)


def _split_edge_w(W, C, pad=0):
    wd = W[:, :C].T
    wc = W[:, C:].T
    if pad:
        zpad = jnp.zeros((pad, W.shape[0]), W.dtype)
        wd = jnp.concatenate([wd, zpad], axis=0)
        wc = jnp.concatenate([wc, zpad], axis=0)
    return wd, wc


def kernel(x, W1, W2, W3, W4, W5, W6, W8, W9, W10, W11):
    B = x.shape[0]
    xt = jnp.transpose(x, (0, 2, 1))
    xp = jnp.concatenate([xt, jnp.zeros((B, N, 5), xt.dtype)], axis=-1)

    wd1, wc1 = _split_edge_w(W1, 3, pad=5)
    wd2, wc2 = _split_edge_w(W3, 64)
    wd3, wc3 = _split_edge_w(W5, 64)

    x1 = _edge_stage_sc(xp, wd1, wc1, W2.T)
    x2 = _edge_stage_sc(x1, wd2, wc2, W4.T)
    x3 = _edge_stage_sc(x2, wd3, wc3, None)

    outT = _head(x1, x2, x3,
                 W6[:, :64].T, W6[:, 64:128].T, W6[:, 128:].T,
                 W8[:, :1024].T, W8[:, 1024:1088].T, W8[:, 1088:1152].T,
                 W8[:, 1152:].T, W9.T, W10.T, W11.T)
    return jnp.transpose(outT, (0, 2, 1))
